# baseline (speedup 1.0000x reference)
import functools

import jax
import jax.numpy as jnp
from jax import lax
from jax.experimental import pallas as pl
from jax.experimental.pallas import tpu as pltpu
from jax.experimental.pallas import tpu_sc as plsc

B = 16384
NV = 100
NG = 1000
NW = 32
RW = B // NW
CH = 128
NCH = RW // CH
L = 16
UB = 10
NVB = NV // UB


def _sc_body(gene_hbm, table_hbm, out_hbm,
             table_v, g0_v, g1_v, out_v, sem_t, sem0, sem1):
    wid = lax.axis_index("s") * 2 + lax.axis_index("c")
    base_row = wid * RW

    bufs = (g0_v, g1_v)
    sems = (sem0, sem1)

    def gene_copy(c):
        return pltpu.make_async_copy(
            gene_hbm.at[:, pl.ds(base_row + c * CH, CH)],
            bufs[c % 2], sems[c % 2])

    cp = gene_copy(0)
    cp.start()

    slice_cps = [
        pltpu.make_async_copy(
            table_hbm.at[pl.ds(vb * UB * NG, UB * NG)],
            table_v.at[pl.ds(vb * UB * NG, UB * NG)], sem_t)
        for vb in range(NVB)
    ]
    for scp in slice_cps:
        scp.start()

    fzero = jnp.zeros((L,), jnp.float32)
    cp.wait()
    cp = gene_copy(1)
    cp.start()

    gbuf = bufs[0]
    for vb in range(NVB):
        slice_cps[vb].wait()

        def slice_body(i0, _, vb=vb):
            col = i0 * L
            acc = fzero if vb == 0 else out_v[pl.ds(col, L)]
            for j in range(UB):
                v = vb * UB + j
                g = gbuf[v, pl.ds(col, L)]
                acc = acc + plsc.load_gather(table_v, [g + v * NG])
            out_v[pl.ds(col, L)] = acc
            return 0

        lax.fori_loop(0, CH // L, slice_body, 0)

    for c in range(1, NCH):
        cp.wait()
        if c + 1 < NCH:
            cp = gene_copy(c + 1)
            cp.start()
        gbuf = bufs[c % 2]

        def group_body(i0, _):
            col = i0 * L

            def blk(b, carry):
                acc, prev = carry
                new = []
                for j in range(UB):
                    v = b * UB + j
                    g = gbuf[v, pl.ds(col, L)]
                    new.append(plsc.load_gather(table_v, [g + v * NG]))
                for x in prev:
                    acc = acc + x
                return acc, tuple(new)

            acc, last = lax.fori_loop(0, NV // UB, blk,
                                      (fzero, (fzero,) * UB))
            for x in last:
                acc = acc + x
            out_v[pl.ds(c * CH + col, L)] = acc
            return 0

        lax.fori_loop(0, CH // L, group_body, 0)

    pltpu.sync_copy(out_v, out_hbm.at[pl.ds(base_row, RW)])


@jax.jit
def kernel(gene, genes):
    gene_t = gene.astype(jnp.int32).T
    table_flat = genes.reshape(-1).astype(jnp.float32)

    sc_call = functools.partial(
        pl.kernel,
        mesh=plsc.VectorSubcoreMesh(core_axis_name="c", subcore_axis_name="s"),
        out_type=jax.ShapeDtypeStruct((B,), jnp.float32),
        scratch_types=[
            pltpu.VMEM((NV * NG,), jnp.float32),
            pltpu.VMEM((NV, CH), jnp.int32),
            pltpu.VMEM((NV, CH), jnp.int32),
            pltpu.VMEM((RW,), jnp.float32),
            pltpu.SemaphoreType.DMA,
            pltpu.SemaphoreType.DMA,
            pltpu.SemaphoreType.DMA,
        ],
        compiler_params=pltpu.CompilerParams(needs_layout_passes=False),
    )(_sc_body)

    fit = sc_call(gene_t, table_flat)
    return fit.reshape(B, 1)


# speedup vs baseline: 1.0667x; 1.0667x over previous
import functools

import jax
import jax.numpy as jnp
from jax import lax
from jax.experimental import pallas as pl
from jax.experimental.pallas import tpu as pltpu
from jax.experimental.pallas import tpu_sc as plsc

B = 16384
NV = 100
NVA = 48
NVB = NV - NVA
NG = 1000
NW = 32
RW = B // NW
PR = 2 * RW
CH = 128
NCH = PR // CH
L = 16


def _sc_body(gene_hbm, table_hbm, out_hbm,
             table_v, g0a_v, g1a_v, g0b_v, g1b_v, part_v, shared_s,
             sem_t, sem0, sem1, sem_x):
    cid = lax.axis_index("c")
    sid = lax.axis_index("s")
    wid = sid * 2 + cid
    wid_p = (sid ^ 1) * 2 + cid
    base_own = wid * RW
    base_par = wid_p * RW

    fzero = jnp.zeros((L,), jnp.float32)

    def chunk_row(c):
        if c < NCH // 2:
            return base_own + c * CH
        return base_par + (c - NCH // 2) * CH

    def compute(v0, nv, ub, bufs, sems):
        tbl_cp = pltpu.make_async_copy(
            table_hbm.at[pl.ds(v0 * NG, nv * NG)],
            table_v.at[pl.ds(0, nv * NG)], sem_t)
        tbl_cp.start()

        def gene_copy(c):
            return pltpu.make_async_copy(
                gene_hbm.at[pl.ds(v0, nv), pl.ds(chunk_row(c), CH)],
                bufs[c % 2], sems[c % 2])

        cp = gene_copy(0)
        cp.start()
        tbl_cp.wait()

        for c in range(NCH):
            cp.wait()
            if c + 1 < NCH:
                cp = gene_copy(c + 1)
                cp.start()
            gbuf = bufs[c % 2]

            def group_body(i0, _):
                col = i0 * L

                def blk(b, carry):
                    acc, prev = carry
                    new = []
                    for j in range(ub):
                        v = b * ub + j
                        g = gbuf[v, pl.ds(col, L)]
                        new.append(
                            plsc.load_gather(table_v, [g + v * NG]))
                    for x in prev:
                        acc = acc + x
                    return acc, tuple(new)

                acc, last = lax.fori_loop(0, nv // ub, blk,
                                          (fzero, (fzero,) * ub))
                for x in last:
                    acc = acc + x
                part_v[pl.ds(c * CH + col, L)] = acc
                return 0

            lax.fori_loop(0, CH // L, group_body, 0)

    @pl.when(sid % 2 == 0)
    def _even():
        compute(0, NVA, 8, (g0a_v, g1a_v), (sem0, sem1))

    @pl.when(sid % 2 == 1)
    def _odd():
        compute(NVA, NVB, 13, (g0b_v, g1b_v), (sem0, sem1))

    sid_p = sid ^ 1
    pltpu.sync_copy(part_v.at[pl.ds(RW, RW)],
                    shared_s.at[pl.ds(sid_p * RW, RW)])
    plsc.subcore_barrier()
    xcp = pltpu.make_async_copy(shared_s.at[pl.ds(sid * RW, RW)],
                                part_v.at[pl.ds(RW, RW)], sem_x)
    xcp.start()
    xcp.wait()

    def add_body(i, _):
        off = i * L
        part_v[pl.ds(off, L)] = (part_v[pl.ds(off, L)]
                                 + part_v[pl.ds(RW + off, L)])
        return 0

    lax.fori_loop(0, RW // L, add_body, 0)

    pltpu.sync_copy(part_v.at[pl.ds(0, RW)],
                    out_hbm.at[pl.ds(base_own, RW)])


@jax.jit
def kernel(gene, genes):
    gene_t = gene.astype(jnp.int32).T
    table_flat = genes.reshape(-1).astype(jnp.float32)

    sc_call = functools.partial(
        pl.kernel,
        mesh=plsc.VectorSubcoreMesh(core_axis_name="c", subcore_axis_name="s"),
        out_type=jax.ShapeDtypeStruct((B,), jnp.float32),
        scratch_types=[
            pltpu.VMEM((NVB * NG,), jnp.float32),
            pltpu.VMEM((NVA, CH), jnp.int32),
            pltpu.VMEM((NVA, CH), jnp.int32),
            pltpu.VMEM((NVB, CH), jnp.int32),
            pltpu.VMEM((NVB, CH), jnp.int32),
            pltpu.VMEM((PR,), jnp.float32),
            pltpu.VMEM_SHARED((16 * RW,), jnp.float32),
            pltpu.SemaphoreType.DMA,
            pltpu.SemaphoreType.DMA,
            pltpu.SemaphoreType.DMA,
            pltpu.SemaphoreType.DMA,
        ],
        compiler_params=pltpu.CompilerParams(needs_layout_passes=False),
    )(_sc_body)

    fit = sc_call(gene_t, table_flat)
    return fit.reshape(B, 1)
